# Initial kernel scaffold; baseline (speedup 1.0000x reference)
import jax
import jax.numpy as jnp
from jax.experimental import pallas as pl


def kernel(x, edge_index, W1, b1, W2, b2):
    raise NotImplementedError("write your pallas kernel here")


# speedup vs baseline: 8.2969x; 8.2969x over previous
import functools
import jax
import jax.numpy as jnp
from jax import lax
from jax.experimental import pallas as pl
from jax.experimental.pallas import tpu as pltpu
from jax.experimental.pallas import tpu_sc as plsc

N = 10000
E = 320000
IN_CH = 128
HID_CH = 128
OUT_CH = 64

NC = 2
NS = 16
NW = NC * NS
CHUNK = 128
NCHUNK = 80
EPW = NCHUNK * CHUNK
EPAD = NW * EPW
NT = 10112
RPW = NT // NS
DEGW = 128

_mesh = plsc.VectorSubcoreMesh(core_axis_name="c", subcore_axis_name="s")


def _deg_body(dst_hbm, ones_hbm, zeros_hbm, degp_hbm, dst_v, ones_v, table, sem):
    c = lax.axis_index("c")
    s = lax.axis_index("s")
    w = s * NC + c
    pltpu.sync_copy(dst_hbm.at[w], dst_v)
    pltpu.sync_copy(ones_hbm, ones_v)
    pltpu.sync_copy(zeros_hbm.at[pl.ds(s * RPW, RPW)], table.at[pl.ds(s * RPW, RPW)])
    plsc.subcore_barrier()

    def body(j, carry):
        pltpu.sync_copy(ones_v, table.at[dst_v.at[j]], add=True)
        return carry

    lax.fori_loop(0, NCHUNK, body, 0)
    plsc.subcore_barrier()
    pltpu.sync_copy(table.at[pl.ds(s * RPW, RPW)], degp_hbm.at[c, pl.ds(s * RPW, RPW)])


_deg_kernel = pl.kernel(
    _deg_body,
    out_type=jax.ShapeDtypeStruct((NC, NT, DEGW), jnp.float32),
    mesh=_mesh,
    scratch_types=[
        pltpu.VMEM((NCHUNK, CHUNK), jnp.int32),
        pltpu.VMEM((CHUNK, DEGW), jnp.float32),
        pltpu.VMEM_SHARED((NT, DEGW), jnp.float32),
        pltpu.SemaphoreType.DMA,
    ],
)


def _scat_body(g_hbm, src_hbm, dst_hbm, zeros_hbm, out_hbm, src_v, dst_v, rows_v,
               table, sem):
    c = lax.axis_index("c")
    s = lax.axis_index("s")
    w = s * NC + c
    pltpu.sync_copy(src_hbm.at[w], src_v)
    pltpu.sync_copy(dst_hbm.at[w], dst_v)
    pltpu.sync_copy(zeros_hbm.at[pl.ds(s * RPW, RPW)], table.at[pl.ds(s * RPW, RPW)])
    plsc.subcore_barrier()

    def body(j, carry):
        pltpu.async_copy(g_hbm.at[src_v.at[j]], rows_v, sem).wait()
        pltpu.sync_copy(rows_v, table.at[dst_v.at[j]], add=True)
        return carry

    lax.fori_loop(0, NCHUNK, body, 0)
    plsc.subcore_barrier()
    pltpu.sync_copy(table.at[pl.ds(s * RPW, RPW)], out_hbm.at[c, pl.ds(s * RPW, RPW)])


def _make_scatter(ch):
    return pl.kernel(
        _scat_body,
        out_type=jax.ShapeDtypeStruct((NC, NT, ch), jnp.float32),
        mesh=_mesh,
        scratch_types=[
            pltpu.VMEM((NCHUNK, CHUNK), jnp.int32),
            pltpu.VMEM((NCHUNK, CHUNK), jnp.int32),
            pltpu.VMEM((CHUNK, ch), jnp.float32),
            pltpu.VMEM_SHARED((NT, ch), jnp.float32),
            pltpu.SemaphoreType.DMA,
        ],
    )


_scatter128 = _make_scatter(HID_CH)

ROWB = 1000


def _dinv(degp_ref):
    degsum = degp_ref[0] + degp_ref[1]
    deg = degsum[:, 0:1] + 1.0
    return lax.rsqrt(deg)


def _pre_body(x_ref, w_ref, degp_ref, g_ref):
    dinv = _dinv(degp_ref)
    g_ref[...] = jnp.dot(x_ref[...], w_ref[...],
                         preferred_element_type=jnp.float32) * dinv


def _mid_body(sp_ref, g1_ref, degp_ref, b1_ref, w2_ref, g2_ref):
    dinv = _dinv(degp_ref)
    ssum = sp_ref[0] + sp_ref[1]
    h = jnp.maximum(dinv * (ssum + g1_ref[...]) + b1_ref[...], 0.0)
    g2_ref[...] = jnp.dot(h, w2_ref[...],
                          preferred_element_type=jnp.float32) * dinv


def _fin_body(sp_ref, g2_ref, degp_ref, b2_ref, z_ref):
    dinv = _dinv(degp_ref)
    ssum = sp_ref[0] + sp_ref[1]
    z = jnp.maximum(dinv * (ssum + g2_ref[...]) + b2_ref[...], 0.0)
    z_ref[...] = z[:, :OUT_CH]


_GRID = (N // ROWB,)


def _rows(ch):
    return pl.BlockSpec((ROWB, ch), lambda i: (i, 0))


def _part(ch):
    return pl.BlockSpec((NC, ROWB, ch), lambda i: (0, i, 0))


def _full(r, ch):
    return pl.BlockSpec((r, ch), lambda i: (0, 0))


_pre = pl.pallas_call(
    _pre_body,
    grid=_GRID,
    in_specs=[_rows(IN_CH), _full(IN_CH, HID_CH), _part(DEGW)],
    out_specs=_rows(HID_CH),
    out_shape=jax.ShapeDtypeStruct((N, HID_CH), jnp.float32),
)

_mid = pl.pallas_call(
    _mid_body,
    grid=_GRID,
    in_specs=[_part(HID_CH), _rows(HID_CH), _part(DEGW), _full(1, HID_CH),
              _full(HID_CH, HID_CH)],
    out_specs=_rows(HID_CH),
    out_shape=jax.ShapeDtypeStruct((N, HID_CH), jnp.float32),
)

_fin = pl.pallas_call(
    _fin_body,
    grid=_GRID,
    in_specs=[_part(HID_CH), _rows(HID_CH), _part(DEGW), _full(1, HID_CH)],
    out_specs=_rows(OUT_CH),
    out_shape=jax.ShapeDtypeStruct((N, OUT_CH), jnp.float32),
)


@jax.jit
def _run(x, edge_index, W1, b1, W2, b2):
    ei = edge_index.astype(jnp.int32)
    src = jnp.concatenate([ei[0], jnp.zeros((EPAD - E,), jnp.int32)])
    dst = jnp.concatenate([ei[1], jnp.full((EPAD - E,), N, jnp.int32)])
    src3 = src.reshape(NW, NCHUNK, CHUNK)
    dst3 = dst.reshape(NW, NCHUNK, CHUNK)

    ones_deg = jnp.ones((CHUNK, DEGW), jnp.float32)
    zeros_deg = jnp.zeros((NT, DEGW), jnp.float32)
    zeros128 = jnp.zeros((NT, HID_CH), jnp.float32)
    W2p = jnp.pad(W2, ((0, 0), (0, HID_CH - OUT_CH)))
    b2p = jnp.pad(b2, (0, HID_CH - OUT_CH))

    degp = _deg_kernel(dst3, ones_deg, zeros_deg)

    g1 = _pre(x, W1, degp)
    s1 = _scatter128(g1, src3, dst3, zeros128)
    g2 = _mid(s1, g1, degp, b1.reshape(1, HID_CH), W2p)
    s2 = _scatter128(g2, src3, dst3, zeros128)
    z = _fin(s2, g2, degp, b2p.reshape(1, HID_CH))
    return z


def kernel(x, edge_index, W1, b1, W2, b2):
    return _run(x, edge_index, W1, b1, W2, b2)
